# Initial kernel scaffold; baseline (speedup 1.0000x reference)
import jax
import jax.numpy as jnp
from jax.experimental import pallas as pl


def kernel(idx_i, idx_j, precomputed_features, feature_weights, intercept):
    raise NotImplementedError("write your pallas kernel here")


# speedup vs baseline: 1.2877x; 1.2877x over previous
import functools

import jax
import jax.numpy as jnp
from jax import lax
from jax.experimental import pallas as pl
from jax.experimental.pallas import tpu as pltpu
from jax.experimental.pallas import tpu_sc as plsc

_L = 16
_NC = 2
_NS = 16
_NW = _NC * _NS
_C = 32


def _fastrp_body(npaths, npow, nrows, dim, npairs,
                 params_hbm, ii_hbm, jj_hbm, flat_hbm, out_hbm,
                 w_v, ii_v, jj_v, offs_v, rows_v, out_v, sem):
    pk = npaths * npow
    wid = lax.axis_index("s") * _NC + lax.axis_index("c")
    b_per_w = npairs // _NW
    n_sub = b_per_w // _C
    base = wid * b_per_w

    pltpu.sync_copy(params_hbm, w_v)
    v = w_v[...]
    lane = lax.iota(jnp.int32, _L)
    e = jnp.exp(v)
    w = jnp.zeros((_L,), jnp.float32)
    for p in range(npaths):
        m = (lane >= p * npow) & (lane < (p + 1) * npow)
        s = jnp.sum(jnp.where(m, e, 0.0))
        w = jnp.where(m, e / s, w)
    wsc = [w[c] for c in range(pk)]
    b0 = v[pk]

    def sub_body(s, carry):
        cbase = pl.multiple_of(base + s * _C, _C)
        pltpu.sync_copy(ii_hbm.at[pl.ds(cbase, _C)], ii_v)
        pltpu.sync_copy(jj_hbm.at[pl.ds(cbase, _C)], jj_v)
        for q in range(_C // _L):
            vi = ii_v[pl.ds(q * _L, _L)]
            vj = jj_v[pl.ds(q * _L, _L)]
            for c in range(pk):
                offs_v[c, pl.ds(q * _L, _L)] = vi + c * nrows
                offs_v[pk + c, pl.ds(q * _L, _L)] = vj + c * nrows
        cps = [pltpu.async_copy(flat_hbm.at[offs_v.at[c]], rows_v.at[c], sem)
               for c in range(2 * pk)]
        for cp in cps:
            cp.wait()

        inv_dim = 1.0 / dim
        for grp in range(_C // _L):

            def pair_body(t, z):
                tt = grp * _L + t
                acc = jnp.zeros((_L,), jnp.float32)
                for g in range(dim // _L):
                    d = None
                    for c in range(pk):
                        ri = rows_v[c, tt, pl.ds(g * _L, _L)]
                        rj = rows_v[pk + c, tt, pl.ds(g * _L, _L)]
                        dc = wsc[c] * (ri - rj)
                        d = dc if d is None else d + dc
                    acc = acc + d * d
                return jnp.where(lane == t, jnp.sum(acc), z)

            z = lax.fori_loop(0, _L, pair_body, jnp.zeros((_L,), jnp.float32))
            logit = b0 - z * inv_dim
            out_v[pl.ds(grp * _L, _L)] = 1.0 / (1.0 + jnp.exp(-logit))
        pltpu.sync_copy(out_v, out_hbm.at[pl.ds(cbase, _C)])
        return carry

    lax.fori_loop(0, n_sub, sub_body, 0)


def kernel(idx_i, idx_j, precomputed_features, feature_weights, intercept):
    npaths, npow, nrows, dim = precomputed_features.shape
    npairs = idx_i.shape[0]
    pk = npaths * npow
    flat = precomputed_features.reshape(pk * nrows, dim)
    params = jnp.concatenate([
        feature_weights.reshape(-1).astype(jnp.float32),
        jnp.reshape(intercept, (1,)).astype(jnp.float32),
        jnp.zeros((_L - pk - 1,), jnp.float32),
    ])
    ii = idx_i.astype(jnp.int32)
    jj = idx_j.astype(jnp.int32)

    mesh = plsc.VectorSubcoreMesh(core_axis_name="c", subcore_axis_name="s",
                                  num_cores=_NC, num_subcores=_NS)
    body = functools.partial(_fastrp_body, npaths, npow, nrows, dim, npairs)
    f = pl.kernel(
        body,
        out_type=jax.ShapeDtypeStruct((npairs,), jnp.float32),
        mesh=mesh,
        compiler_params=pltpu.CompilerParams(needs_layout_passes=False),
        scratch_types=[
            pltpu.VMEM((_L,), jnp.float32),
            pltpu.VMEM((_C,), jnp.int32),
            pltpu.VMEM((_C,), jnp.int32),
            pltpu.VMEM((2 * pk, _C), jnp.int32),
            pltpu.VMEM((2 * pk, _C, dim), jnp.float32),
            pltpu.VMEM((_C,), jnp.float32),
            pltpu.SemaphoreType.DMA,
        ],
    )
    return f(params, ii, jj, flat)


# speedup vs baseline: 1.5915x; 1.2360x over previous
import functools

import jax
import jax.numpy as jnp
from jax import lax
from jax.experimental import pallas as pl
from jax.experimental.pallas import tpu as pltpu
from jax.experimental.pallas import tpu_sc as plsc

_L = 16
_NC = 2
_NS = 16
_NW = _NC * _NS
_C = 32


def _fastrp_body(npaths, npow, nrows, dim, npairs,
                 params_hbm, ii_hbm, jj_hbm, flat_hbm, out_hbm,
                 w_v, ii0, jj0, offs0, rows0, ii1, jj1, offs1, rows1,
                 out_v, sem0, sem1):
    pk = npaths * npow
    wid = lax.axis_index("s") * _NC + lax.axis_index("c")
    b_per_w = npairs // _NW
    n_sub = b_per_w // _C
    base = wid * b_per_w

    pltpu.sync_copy(params_hbm, w_v)
    v = w_v[...]
    lane = lax.iota(jnp.int32, _L)
    e = jnp.exp(v)
    w = jnp.zeros((_L,), jnp.float32)
    for p in range(npaths):
        m = (lane >= p * npow) & (lane < (p + 1) * npow)
        s = jnp.sum(jnp.where(m, e, 0.0))
        w = jnp.where(m, e / s, w)
    wsc = [w[c] for c in range(pk)]
    b0 = v[pk]
    inv_dim = 1.0 / dim

    def prefetch(cbase, ii_v, jj_v, offs_v, rows_v, sem):
        pltpu.sync_copy(ii_hbm.at[pl.ds(cbase, _C)], ii_v)
        pltpu.sync_copy(jj_hbm.at[pl.ds(cbase, _C)], jj_v)
        for q in range(_C // _L):
            vi = ii_v[pl.ds(q * _L, _L)]
            vj = jj_v[pl.ds(q * _L, _L)]
            for c in range(pk):
                offs_v[c, pl.ds(q * _L, _L)] = vi + c * nrows
                offs_v[pk + c, pl.ds(q * _L, _L)] = vj + c * nrows
        for c in range(2 * pk):
            pltpu.async_copy(flat_hbm.at[offs_v.at[c]], rows_v.at[c], sem)

    def process(cbase, offs_v, rows_v, sem):
        for c in range(2 * pk):
            pltpu.make_async_copy(
                flat_hbm.at[offs_v.at[c]], rows_v.at[c], sem).wait()
        for grp in range(_C // _L):

            def pair_body(t, z):
                tt = grp * _L + t
                acc = jnp.zeros((_L,), jnp.float32)
                for g in range(dim // _L):
                    d = None
                    for c in range(pk):
                        ri = rows_v[c, tt, pl.ds(g * _L, _L)]
                        rj = rows_v[pk + c, tt, pl.ds(g * _L, _L)]
                        dc = wsc[c] * (ri - rj)
                        d = dc if d is None else d + dc
                    acc = acc + d * d
                return jnp.where(lane == t, jnp.sum(acc), z)

            z = lax.fori_loop(0, _L, pair_body, jnp.zeros((_L,), jnp.float32))
            logit = b0 - z * inv_dim
            out_v[pl.ds(grp * _L, _L)] = 1.0 / (1.0 + jnp.exp(-logit))
        pltpu.sync_copy(out_v, out_hbm.at[pl.ds(cbase, _C)])

    prefetch(base, ii0, jj0, offs0, rows0, sem0)

    def sub_body(s2, carry):
        c0 = pl.multiple_of(base + (2 * s2) * _C, _C)
        c1 = pl.multiple_of(base + (2 * s2 + 1) * _C, _C)
        prefetch(c1, ii1, jj1, offs1, rows1, sem1)
        process(c0, offs0, rows0, sem0)

        @pl.when(2 * s2 + 2 < n_sub)
        def _():
            c2 = pl.multiple_of(base + (2 * s2 + 2) * _C, _C)
            prefetch(c2, ii0, jj0, offs0, rows0, sem0)

        process(c1, offs1, rows1, sem1)
        return carry

    lax.fori_loop(0, n_sub // 2, sub_body, 0)


def kernel(idx_i, idx_j, precomputed_features, feature_weights, intercept):
    npaths, npow, nrows, dim = precomputed_features.shape
    npairs = idx_i.shape[0]
    pk = npaths * npow
    flat = precomputed_features.reshape(pk * nrows, dim)
    params = jnp.concatenate([
        feature_weights.reshape(-1).astype(jnp.float32),
        jnp.reshape(intercept, (1,)).astype(jnp.float32),
        jnp.zeros((_L - pk - 1,), jnp.float32),
    ])
    ii = idx_i.astype(jnp.int32)
    jj = idx_j.astype(jnp.int32)

    mesh = plsc.VectorSubcoreMesh(core_axis_name="c", subcore_axis_name="s",
                                  num_cores=_NC, num_subcores=_NS)
    body = functools.partial(_fastrp_body, npaths, npow, nrows, dim, npairs)
    dbuf = [
        pltpu.VMEM((_C,), jnp.int32),
        pltpu.VMEM((_C,), jnp.int32),
        pltpu.VMEM((2 * pk, _C), jnp.int32),
        pltpu.VMEM((2 * pk, _C, dim), jnp.float32),
    ]
    f = pl.kernel(
        body,
        out_type=jax.ShapeDtypeStruct((npairs,), jnp.float32),
        mesh=mesh,
        compiler_params=pltpu.CompilerParams(needs_layout_passes=False),
        scratch_types=(
            [pltpu.VMEM((_L,), jnp.float32)]
            + dbuf + dbuf
            + [pltpu.VMEM((_C,), jnp.float32),
               pltpu.SemaphoreType.DMA,
               pltpu.SemaphoreType.DMA]
        ),
    )
    return f(params, ii, jj, flat)


# speedup vs baseline: 1.8057x; 1.1346x over previous
import functools

import jax
import jax.numpy as jnp
from jax import lax
from jax.experimental import pallas as pl
from jax.experimental.pallas import tpu as pltpu
from jax.experimental.pallas import tpu_sc as plsc

_L = 16
_NC = 2
_NS = 16
_NW = _NC * _NS
_C = 32


def _fastrp_body(npaths, npow, nrows, dim, npairs,
                 params_hbm, ii_hbm, jj_hbm, flat_hbm, out_hbm,
                 w_v, ii_all, jj_all, offs0, offs1, rows0, rows1,
                 out_all, sem0, sem1):
    pk = npaths * npow
    wid = lax.axis_index("s") * _NC + lax.axis_index("c")
    b_per_w = npairs // _NW
    n_sub = b_per_w // _C
    base = wid * b_per_w

    pltpu.sync_copy(params_hbm, w_v)
    v = w_v[...]
    lane = lax.iota(jnp.int32, _L)
    e = jnp.exp(v)
    w = jnp.zeros((_L,), jnp.float32)
    for p in range(npaths):
        m = (lane >= p * npow) & (lane < (p + 1) * npow)
        s = jnp.sum(jnp.where(m, e, 0.0))
        w = jnp.where(m, e / s, w)
    wsc = [w[c] for c in range(pk)]
    b0 = v[pk]
    inv_dim = 1.0 / dim

    pltpu.sync_copy(ii_hbm.at[pl.ds(base, b_per_w)], ii_all)
    pltpu.sync_copy(jj_hbm.at[pl.ds(base, b_per_w)], jj_all)

    def prefetch(s, offs_v, rows_v, sem):
        for q in range(_C // _L):
            o = s * _C + q * _L
            vi = ii_all[pl.ds(o, _L)]
            vj = jj_all[pl.ds(o, _L)]
            for c in range(pk):
                offs_v[c, pl.ds(q * _L, _L)] = vi + c * nrows
                offs_v[pk + c, pl.ds(q * _L, _L)] = vj + c * nrows
        for c in range(2 * pk):
            pltpu.async_copy(
                flat_hbm.at[offs_v.at[c]], rows_v.at[c], sem)

    def process(s, offs_v, rows_v, sem):
        for c in range(2 * pk):
            pltpu.make_async_copy(
                flat_hbm.at[offs_v.at[c]], rows_v.at[c], sem).wait()
        for grp in range(_C // _L):

            def pair_body(t, z):
                tt = grp * _L + t
                acc = jnp.zeros((_L,), jnp.float32)
                for g in range(dim // _L):
                    d = None
                    for c in range(pk):
                        ri = rows_v[c, tt, pl.ds(g * _L, _L)]
                        rj = rows_v[pk + c, tt, pl.ds(g * _L, _L)]
                        dc = wsc[c] * (ri - rj)
                        d = dc if d is None else d + dc
                    acc = acc + d * d
                return jnp.where(lane == t, jnp.sum(acc), z)

            z = lax.fori_loop(0, _L, pair_body, jnp.zeros((_L,), jnp.float32))
            logit = b0 - z * inv_dim
            out_all[pl.ds(s * _C + grp * _L, _L)] = 1.0 / (1.0 + jnp.exp(-logit))

    prefetch(0, offs0, rows0, sem0)

    def sub_body(s2, carry):
        s0 = 2 * s2
        prefetch(s0 + 1, offs1, rows1, sem1)
        process(s0, offs0, rows0, sem0)

        @pl.when(s0 + 2 < n_sub)
        def _():
            prefetch(s0 + 2, offs0, rows0, sem0)

        process(s0 + 1, offs1, rows1, sem1)
        return carry

    lax.fori_loop(0, n_sub // 2, sub_body, 0)
    pltpu.sync_copy(out_all, out_hbm.at[pl.ds(base, b_per_w)])


def kernel(idx_i, idx_j, precomputed_features, feature_weights, intercept):
    npaths, npow, nrows, dim = precomputed_features.shape
    npairs = idx_i.shape[0]
    pk = npaths * npow
    b_per_w = npairs // _NW
    n_sub = b_per_w // _C
    flat = precomputed_features.reshape(pk * nrows, dim)
    params = jnp.concatenate([
        feature_weights.reshape(-1).astype(jnp.float32),
        jnp.reshape(intercept, (1,)).astype(jnp.float32),
        jnp.zeros((_L - pk - 1,), jnp.float32),
    ])
    ii = idx_i.astype(jnp.int32)
    jj = idx_j.astype(jnp.int32)

    mesh = plsc.VectorSubcoreMesh(core_axis_name="c", subcore_axis_name="s",
                                  num_cores=_NC, num_subcores=_NS)
    body = functools.partial(_fastrp_body, npaths, npow, nrows, dim, npairs)
    f = pl.kernel(
        body,
        out_type=jax.ShapeDtypeStruct((npairs,), jnp.float32),
        mesh=mesh,
        compiler_params=pltpu.CompilerParams(needs_layout_passes=False),
        scratch_types=[
            pltpu.VMEM((_L,), jnp.float32),
            pltpu.VMEM((b_per_w,), jnp.int32),
            pltpu.VMEM((b_per_w,), jnp.int32),
            pltpu.VMEM((2 * pk, _C), jnp.int32),
            pltpu.VMEM((2 * pk, _C), jnp.int32),
            pltpu.VMEM((2 * pk, _C, dim), jnp.float32),
            pltpu.VMEM((2 * pk, _C, dim), jnp.float32),
            pltpu.VMEM((b_per_w,), jnp.float32),
            pltpu.SemaphoreType.DMA,
            pltpu.SemaphoreType.DMA,
        ],
    )
    return f(params, ii, jj, flat)


# speedup vs baseline: 2.1287x; 1.1789x over previous
import functools

import jax
import jax.numpy as jnp
from jax import lax
from jax.experimental import pallas as pl
from jax.experimental.pallas import tpu as pltpu
from jax.experimental.pallas import tpu_sc as plsc

_L = 16
_NC = 2
_NS = 16
_NW = _NC * _NS
_C = 32


def _fastrp_body(npaths, npow, nrows, dim, npairs,
                 params_hbm, ii_hbm, jj_hbm, flat_hbm, out_hbm,
                 w_v, ii_all, jj_all, offs0, offs1, rows0, rows1,
                 out_all, sem0, sem1):
    pk = npaths * npow
    wid = lax.axis_index("s") * _NC + lax.axis_index("c")
    b_per_w = npairs // _NW
    n_sub = b_per_w // _C
    base = wid * b_per_w

    pltpu.sync_copy(params_hbm, w_v)
    v = w_v[...]
    lane = lax.iota(jnp.int32, _L)
    e = jnp.exp(v)
    w = jnp.zeros((_L,), jnp.float32)
    for p in range(npaths):
        m = (lane >= p * npow) & (lane < (p + 1) * npow)
        s = jnp.sum(jnp.where(m, e, 0.0))
        w = jnp.where(m, e / s, w)
    wsc = [w[c] for c in range(pk)]
    b0 = v[pk]
    inv_dim = 1.0 / dim

    pltpu.sync_copy(ii_hbm.at[pl.ds(base, b_per_w)], ii_all)
    pltpu.sync_copy(jj_hbm.at[pl.ds(base, b_per_w)], jj_all)

    n_str = (2 * pk * _C) // 128

    def prefetch(s, offs_v, rows_v, sem):
        for q in range(_C // _L):
            o = s * _C + q * _L
            vi = ii_all[pl.ds(o, _L)]
            vj = jj_all[pl.ds(o, _L)]
            for c in range(pk):
                ui = c * _C + q * _L
                uj = (pk + c) * _C + q * _L
                offs_v[ui // 128, pl.ds(ui % 128, _L)] = vi + c * nrows
                offs_v[uj // 128, pl.ds(uj % 128, _L)] = vj + c * nrows
        for g in range(n_str):
            pltpu.async_copy(flat_hbm.at[offs_v.at[g]],
                             rows_v.at[pl.ds(g * 128, 128)], sem)

    def process(s, offs_v, rows_v, sem):
        for g in range(n_str):
            pltpu.make_async_copy(flat_hbm.at[offs_v.at[g]],
                                  rows_v.at[pl.ds(g * 128, 128)], sem).wait()
        for grp in range(_C // _L):

            def pair_body(t, z):
                tt = grp * _L + t
                acc = jnp.zeros((_L,), jnp.float32)
                for g in range(dim // _L):
                    d = None
                    for c in range(pk):
                        ri = rows_v[c * _C + tt, pl.ds(g * _L, _L)]
                        rj = rows_v[(pk + c) * _C + tt, pl.ds(g * _L, _L)]
                        dc = wsc[c] * (ri - rj)
                        d = dc if d is None else d + dc
                    acc = acc + d * d
                return jnp.where(lane == t, jnp.sum(acc), z)

            z = lax.fori_loop(0, _L, pair_body, jnp.zeros((_L,), jnp.float32), unroll=2)
            logit = b0 - z * inv_dim
            out_all[pl.ds(s * _C + grp * _L, _L)] = 1.0 / (1.0 + jnp.exp(-logit))

    prefetch(0, offs0, rows0, sem0)

    def sub_body(s2, carry):
        s0 = 2 * s2
        prefetch(s0 + 1, offs1, rows1, sem1)
        process(s0, offs0, rows0, sem0)

        @pl.when(s0 + 2 < n_sub)
        def _():
            prefetch(s0 + 2, offs0, rows0, sem0)

        process(s0 + 1, offs1, rows1, sem1)
        return carry

    lax.fori_loop(0, n_sub // 2, sub_body, 0)
    pltpu.sync_copy(out_all, out_hbm.at[pl.ds(base, b_per_w)])


def kernel(idx_i, idx_j, precomputed_features, feature_weights, intercept):
    npaths, npow, nrows, dim = precomputed_features.shape
    npairs = idx_i.shape[0]
    pk = npaths * npow
    b_per_w = npairs // _NW
    n_sub = b_per_w // _C
    flat = precomputed_features.reshape(pk * nrows, dim)
    params = jnp.concatenate([
        feature_weights.reshape(-1).astype(jnp.float32),
        jnp.reshape(intercept, (1,)).astype(jnp.float32),
        jnp.zeros((_L - pk - 1,), jnp.float32),
    ])
    ii = idx_i.astype(jnp.int32)
    jj = idx_j.astype(jnp.int32)

    mesh = plsc.VectorSubcoreMesh(core_axis_name="c", subcore_axis_name="s",
                                  num_cores=_NC, num_subcores=_NS)
    body = functools.partial(_fastrp_body, npaths, npow, nrows, dim, npairs)
    f = pl.kernel(
        body,
        out_type=jax.ShapeDtypeStruct((npairs,), jnp.float32),
        mesh=mesh,
        compiler_params=pltpu.CompilerParams(needs_layout_passes=False),
        scratch_types=[
            pltpu.VMEM((_L,), jnp.float32),
            pltpu.VMEM((b_per_w,), jnp.int32),
            pltpu.VMEM((b_per_w,), jnp.int32),
            pltpu.VMEM((2 * pk * _C // 128, 128), jnp.int32),
            pltpu.VMEM((2 * pk * _C // 128, 128), jnp.int32),
            pltpu.VMEM((2 * pk * _C, dim), jnp.float32),
            pltpu.VMEM((2 * pk * _C, dim), jnp.float32),
            pltpu.VMEM((b_per_w,), jnp.float32),
            pltpu.SemaphoreType.DMA,
            pltpu.SemaphoreType.DMA,
        ],
    )
    return f(params, ii, jj, flat)
